# Initial kernel scaffold; baseline (speedup 1.0000x reference)
import jax
import jax.numpy as jnp
from jax.experimental import pallas as pl


def kernel(x, edges, pairs_idx, W0, b0, g0, be0, W1, b1, g1, be1, W2, b2, g2, be2, W3, b3):
    raise NotImplementedError("write your pallas kernel here")


# speedup vs baseline: 3.9289x; 3.9289x over previous
import functools

import jax
import jax.numpy as jnp
from jax import lax
from jax.experimental import pallas as pl
from jax.experimental.pallas import tpu as pltpu
from jax.experimental.pallas import tpu_sc as plsc

D_MODEL = 128
KSZ = 8
D_EDGE = 16
PAD_VAL = -999.0
_INV_SQRT_K = 1.0 / (KSZ ** 0.5)

_BE = 2000
_CH = 128
_PREC = lax.Precision.HIGHEST


def _shift_lanes(v, s):
    n = v.shape[1]
    if s == 0:
        return v
    if s < 0:
        z = jnp.zeros((v.shape[0], -s), v.dtype)
        return jnp.concatenate([z, v[:, : n + s]], axis=1)
    z = jnp.zeros((v.shape[0], s), v.dtype)
    return jnp.concatenate([v[:, s:], z], axis=1)


def _tc_body(edges_ref, xf_ref, xr_ref,
             w0_ref, b0_ref, g0_ref, be0_ref,
             w1_ref, b1_ref, g1_ref, be1_ref,
             w2_ref, b2_ref, g2_ref, be2_ref,
             w3s_ref, b3s_ref,
             axf_ref, axr_ref):
    h = edges_ref[...]
    for w_ref, b_ref, g_ref, be_ref in (
            (w0_ref, b0_ref, g0_ref, be0_ref),
            (w1_ref, b1_ref, g1_ref, be1_ref),
            (w2_ref, b2_ref, g2_ref, be2_ref)):
        h = jnp.dot(h, w_ref[...], precision=_PREC,
                    preferred_element_type=jnp.float32) + b_ref[...]
        h = jnp.maximum(h, 0.0)
        mu = jnp.mean(h, axis=-1, keepdims=True)
        var = jnp.mean((h - mu) ** 2, axis=-1, keepdims=True)
        v = var + 1e-5
        r = lax.rsqrt(v)
        r = r * (1.5 - 0.5 * v * r * r)
        h = (h - mu) * r * g_ref[...] + be_ref[...]

    xf = xf_ref[...]
    xr = xr_ref[...]
    axf = jnp.zeros_like(xf)
    axr = jnp.zeros_like(xr)
    for k in range(KSZ):
        a_k = (jnp.dot(h, w3s_ref[k], precision=_PREC,
                       preferred_element_type=jnp.float32)
               + b3s_ref[k:k + 1, :]) * _INV_SQRT_K
        s = k - KSZ // 2
        axf = axf + _shift_lanes(xf, s) * a_k
        axr = axr + _shift_lanes(xr, s) * a_k
    mask = edges_ref[:, 0:1] == PAD_VAL
    axf_ref[...] = jnp.where(mask, 0.0, axf)
    axr_ref[...] = jnp.where(mask, 0.0, axr)


def _tc_specs(E, BE):
    nb = E // BE
    full = lambda *dims: pl.BlockSpec(dims, lambda i: (0,) * len(dims))
    in_specs = [
        pl.BlockSpec((BE, D_EDGE), lambda i: (i, 0)),
        pl.BlockSpec((BE, D_MODEL), lambda i: (i, 0)),
        pl.BlockSpec((BE, D_MODEL), lambda i: (i + nb, 0)),
        full(D_EDGE, D_MODEL), full(1, D_MODEL), full(1, D_MODEL), full(1, D_MODEL),
        full(D_MODEL, D_MODEL), full(1, D_MODEL), full(1, D_MODEL), full(1, D_MODEL),
        full(D_MODEL, D_MODEL), full(1, D_MODEL), full(1, D_MODEL), full(1, D_MODEL),
        full(KSZ, D_MODEL, D_MODEL), full(KSZ, D_MODEL),
    ]
    out_specs = [
        pl.BlockSpec((BE, D_MODEL), lambda i: (i, 0)),
        pl.BlockSpec((BE, D_MODEL), lambda i: (i, 0)),
    ]
    out_shape = [
        jax.ShapeDtypeStruct((E, D_MODEL), jnp.float32),
        jax.ShapeDtypeStruct((E, D_MODEL), jnp.float32),
    ]
    return dict(grid=(nb,), in_specs=in_specs, out_specs=out_specs,
                out_shape=out_shape)


def _run_tc(edges2, xs, weights):
    E = edges2.shape[0]
    sp = _tc_specs(E, _BE)
    return pl.pallas_call(
        _tc_body,
        grid=sp["grid"],
        in_specs=sp["in_specs"],
        out_specs=sp["out_specs"],
        out_shape=sp["out_shape"],
    )(edges2, xs, xs, *weights)


def _sc_mesh():
    return plsc.VectorSubcoreMesh(core_axis_name="core",
                                  subcore_axis_name="subcore")


def _sc_gather(x2, src2d):
    M = src2d.shape[1]

    @functools.partial(
        pl.kernel,
        out_type=jax.ShapeDtypeStruct((M, D_MODEL), jnp.float32),
        mesh=_sc_mesh(),
    )
    def k(x_hbm, i_hbm, o_hbm):
        def body(i_vmem, o_vmem):
            pltpu.sync_copy(x_hbm.at[i_vmem.at[0]], o_vmem)

        pltpu.emit_pipeline(
            body,
            grid=(M // _CH,),
            in_specs=[pl.BlockSpec((1, _CH), lambda i: (0, i))],
            out_specs=[pl.BlockSpec((_CH, D_MODEL), lambda i: (i, 0))],
            core_axis_name=("core", "subcore"),
            dimension_semantics=(pltpu.PARALLEL,),
        )(i_hbm, o_hbm)

    return k(x2, src2d)


def _sc_scatter(axf, axr, dstf, dstr, zeros_nd):
    E = axf.shape[0]
    NP = zeros_nd.shape[0]
    n_sub = 16
    rp = NP // n_sub

    @functools.partial(
        pl.kernel,
        out_type=jax.ShapeDtypeStruct((2, NP, D_MODEL), jnp.float32),
        mesh=_sc_mesh(),
        scratch_types=[pltpu.VMEM_SHARED((NP, D_MODEL), jnp.float32)],
    )
    def k(axf_hbm, axr_hbm, dstf_hbm, dstr_hbm, z_hbm, o_hbm, m_sh):
        c = lax.axis_index("core")
        s = lax.axis_index("subcore")
        pltpu.sync_copy(z_hbm.at[pl.ds(s * rp, rp)],
                        m_sh.at[pl.ds(s * rp, rp)])
        plsc.subcore_barrier()

        def body(ax_vmem, i_vmem):
            pltpu.sync_copy(ax_vmem, m_sh.at[i_vmem.at[0]], add=True)

        for ax_hbm, dst_hbm in ((axf_hbm, dstf_hbm), (axr_hbm, dstr_hbm)):
            pltpu.emit_pipeline(
                body,
                grid=(E // _CH,),
                in_specs=[pl.BlockSpec((_CH, D_MODEL), lambda i: (i, 0)),
                          pl.BlockSpec((1, _CH), lambda i: (0, i))],
                out_specs=[],
                core_axis_name=("core", "subcore"),
                dimension_semantics=(pltpu.PARALLEL,),
            )(ax_hbm, dst_hbm)
        plsc.subcore_barrier()
        pltpu.sync_copy(m_sh.at[pl.ds(s * rp, rp)],
                        o_hbm.at[c, pl.ds(s * rp, rp)])

    return k(axf, axr, dstf, dstr, zeros_nd)


def _add_body(p_ref, o_ref):
    o_ref[0] = p_ref[0] + p_ref[1]


def _run_add(partials, N, BN=2000):
    _, NP, D = partials.shape
    return pl.pallas_call(
        _add_body,
        grid=(N // BN,),
        in_specs=[pl.BlockSpec((2, BN, D), lambda i: (0, i, 0))],
        out_specs=pl.BlockSpec((1, BN, D), lambda i: (0, i, 0)),
        out_shape=jax.ShapeDtypeStruct((1, N, D), jnp.float32),
    )(partials)


def kernel(x, edges, pairs_idx,
           W0, b0, g0, be0, W1, b1, g1, be1, W2, b2, g2, be2, W3, b3):
    B, N, D = x.shape
    _, E, _ = edges.shape
    x2 = x[0]
    edges2 = edges[0]
    p = pairs_idx[0]

    src2d = jnp.concatenate([p[:, 1], p[:, 0]]).reshape(1, 2 * E)
    dstf = p[:, 0].reshape(1, E)
    dstr = p[:, 1].reshape(1, E)

    w3s = jnp.transpose(W3.reshape(D, D, KSZ), (2, 0, 1))
    b3s = jnp.transpose(b3.reshape(D, KSZ), (1, 0))
    weights = (W0, b0.reshape(1, D), g0.reshape(1, D), be0.reshape(1, D),
               W1, b1.reshape(1, D), g1.reshape(1, D), be1.reshape(1, D),
               W2, b2.reshape(1, D), g2.reshape(1, D), be2.reshape(1, D),
               w3s, b3s)

    NP = ((N + 127) // 128) * 128
    xs = _sc_gather(x2, src2d)
    axf, axr = _run_tc(edges2, xs, weights)
    partials = _sc_scatter(axf, axr, dstf, dstr,
                           jnp.zeros((NP, D), jnp.float32))
    return _run_add(partials, N)


# speedup vs baseline: 6.1436x; 1.5637x over previous
import functools

import jax
import jax.numpy as jnp
from jax import lax
from jax.experimental import pallas as pl
from jax.experimental.pallas import tpu as pltpu
from jax.experimental.pallas import tpu_sc as plsc

D_MODEL = 128
KSZ = 8
D_EDGE = 16
PAD_VAL = -999.0
_INV_SQRT_K = 1.0 / (KSZ ** 0.5)

_BE = 2000
_CH = 128
_PREC = lax.Precision.DEFAULT


def _shift_lanes(v, s):
    n = v.shape[1]
    if s == 0:
        return v
    if s < 0:
        z = jnp.zeros((v.shape[0], -s), v.dtype)
        return jnp.concatenate([z, v[:, : n + s]], axis=1)
    z = jnp.zeros((v.shape[0], s), v.dtype)
    return jnp.concatenate([v[:, s:], z], axis=1)


def _tc_body(edges_ref, xf_ref, xr_ref,
             w0_ref, b0_ref, g0_ref, be0_ref,
             w1_ref, b1_ref, g1_ref, be1_ref,
             w2_ref, b2_ref, g2_ref, be2_ref,
             w3s_ref, b3s_ref,
             axf_ref, axr_ref):
    h = edges_ref[...]
    for w_ref, b_ref, g_ref, be_ref in (
            (w0_ref, b0_ref, g0_ref, be0_ref),
            (w1_ref, b1_ref, g1_ref, be1_ref),
            (w2_ref, b2_ref, g2_ref, be2_ref)):
        h = jnp.dot(h, w_ref[...], precision=_PREC,
                    preferred_element_type=jnp.float32) + b_ref[...]
        h = jnp.maximum(h, 0.0)
        mu = jnp.mean(h, axis=-1, keepdims=True)
        var = jnp.mean((h - mu) ** 2, axis=-1, keepdims=True)
        v = var + 1e-5
        r = lax.rsqrt(v)
        r = r * (1.5 - 0.5 * v * r * r)
        h = (h - mu) * r * g_ref[...] + be_ref[...]

    xf = xf_ref[...]
    xr = xr_ref[...]
    axf = jnp.zeros_like(xf)
    axr = jnp.zeros_like(xr)
    for k in range(KSZ):
        a_k = (jnp.dot(h, w3s_ref[k], precision=_PREC,
                       preferred_element_type=jnp.float32)
               + b3s_ref[k:k + 1, :]) * _INV_SQRT_K
        s = k - KSZ // 2
        axf = axf + _shift_lanes(xf, s) * a_k
        axr = axr + _shift_lanes(xr, s) * a_k
    mask = edges_ref[:, 0:1] == PAD_VAL
    axf_ref[...] = jnp.where(mask, 0.0, axf)
    axr_ref[...] = jnp.where(mask, 0.0, axr)


def _tc_specs(E, BE):
    nb = E // BE
    full = lambda *dims: pl.BlockSpec(dims, lambda i: (0,) * len(dims))
    in_specs = [
        pl.BlockSpec((BE, D_EDGE), lambda i: (i, 0)),
        pl.BlockSpec((BE, D_MODEL), lambda i: (i, 0)),
        pl.BlockSpec((BE, D_MODEL), lambda i: (i + nb, 0)),
        full(D_EDGE, D_MODEL), full(1, D_MODEL), full(1, D_MODEL), full(1, D_MODEL),
        full(D_MODEL, D_MODEL), full(1, D_MODEL), full(1, D_MODEL), full(1, D_MODEL),
        full(D_MODEL, D_MODEL), full(1, D_MODEL), full(1, D_MODEL), full(1, D_MODEL),
        full(KSZ, D_MODEL, D_MODEL), full(KSZ, D_MODEL),
    ]
    out_specs = [
        pl.BlockSpec((BE, D_MODEL), lambda i: (i, 0)),
        pl.BlockSpec((BE, D_MODEL), lambda i: (i, 0)),
    ]
    out_shape = [
        jax.ShapeDtypeStruct((E, D_MODEL), jnp.float32),
        jax.ShapeDtypeStruct((E, D_MODEL), jnp.float32),
    ]
    return dict(grid=(nb,), in_specs=in_specs, out_specs=out_specs,
                out_shape=out_shape)


def _run_tc(edges2, xs, weights):
    E = edges2.shape[0]
    sp = _tc_specs(E, _BE)
    return pl.pallas_call(
        _tc_body,
        grid=sp["grid"],
        in_specs=sp["in_specs"],
        out_specs=sp["out_specs"],
        out_shape=sp["out_shape"],
    )(edges2, xs, xs, *weights)


def _sc_mesh():
    return plsc.VectorSubcoreMesh(core_axis_name="core",
                                  subcore_axis_name="subcore")


def _sc_gather(x2, src2d):
    M = src2d.shape[1]

    @functools.partial(
        pl.kernel,
        out_type=jax.ShapeDtypeStruct((M, D_MODEL), jnp.float32),
        mesh=_sc_mesh(),
    )
    def k(x_hbm, i_hbm, o_hbm):
        def body(i_vmem, o_vmem):
            pltpu.sync_copy(x_hbm.at[i_vmem.at[0]], o_vmem)

        pltpu.emit_pipeline(
            body,
            grid=(M // _CH,),
            in_specs=[pl.BlockSpec((1, _CH), lambda i: (0, i))],
            out_specs=[pl.BlockSpec((_CH, D_MODEL), lambda i: (i, 0))],
            core_axis_name=("core", "subcore"),
            dimension_semantics=(pltpu.PARALLEL,),
        )(i_hbm, o_hbm)

    return k(x2, src2d)


def _sc_scatter(axf, axr, dstf, dstr, zeros_nd):
    E = axf.shape[0]
    NP = zeros_nd.shape[0]
    n_sub = 16
    rp = NP // n_sub

    @functools.partial(
        pl.kernel,
        out_type=jax.ShapeDtypeStruct((2, NP, D_MODEL), jnp.float32),
        mesh=_sc_mesh(),
        scratch_types=[pltpu.VMEM_SHARED((NP, D_MODEL), jnp.float32)],
    )
    def k(axf_hbm, axr_hbm, dstf_hbm, dstr_hbm, z_hbm, o_hbm, m_sh):
        c = lax.axis_index("core")
        s = lax.axis_index("subcore")
        pltpu.sync_copy(z_hbm.at[pl.ds(s * rp, rp)],
                        m_sh.at[pl.ds(s * rp, rp)])
        plsc.subcore_barrier()

        def body(ax_vmem, i_vmem):
            pltpu.sync_copy(ax_vmem, m_sh.at[i_vmem.at[0]], add=True)

        for ax_hbm, dst_hbm in ((axf_hbm, dstf_hbm), (axr_hbm, dstr_hbm)):
            pltpu.emit_pipeline(
                body,
                grid=(E // _CH,),
                in_specs=[pl.BlockSpec((_CH, D_MODEL), lambda i: (i, 0)),
                          pl.BlockSpec((1, _CH), lambda i: (0, i))],
                out_specs=[],
                core_axis_name=("core", "subcore"),
                dimension_semantics=(pltpu.PARALLEL,),
            )(ax_hbm, dst_hbm)
        plsc.subcore_barrier()
        pltpu.sync_copy(m_sh.at[pl.ds(s * rp, rp)],
                        o_hbm.at[c, pl.ds(s * rp, rp)])

    return k(axf, axr, dstf, dstr, zeros_nd)


def _add_body(p_ref, o_ref):
    o_ref[0] = p_ref[0] + p_ref[1]


def _run_add(partials, N, BN=2000):
    _, NP, D = partials.shape
    return pl.pallas_call(
        _add_body,
        grid=(N // BN,),
        in_specs=[pl.BlockSpec((2, BN, D), lambda i: (0, i, 0))],
        out_specs=pl.BlockSpec((1, BN, D), lambda i: (0, i, 0)),
        out_shape=jax.ShapeDtypeStruct((1, N, D), jnp.float32),
    )(partials)


def kernel(x, edges, pairs_idx,
           W0, b0, g0, be0, W1, b1, g1, be1, W2, b2, g2, be2, W3, b3):
    B, N, D = x.shape
    _, E, _ = edges.shape
    x2 = x[0]
    edges2 = edges[0]
    p = pairs_idx[0]

    src2d = jnp.concatenate([p[:, 1], p[:, 0]]).reshape(1, 2 * E)
    dstf = p[:, 0].reshape(1, E)
    dstr = p[:, 1].reshape(1, E)

    w3s = jnp.transpose(W3.reshape(D, D, KSZ), (2, 0, 1))
    b3s = jnp.transpose(b3.reshape(D, KSZ), (1, 0))
    weights = (W0, b0.reshape(1, D), g0.reshape(1, D), be0.reshape(1, D),
               W1, b1.reshape(1, D), g1.reshape(1, D), be1.reshape(1, D),
               W2, b2.reshape(1, D), g2.reshape(1, D), be2.reshape(1, D),
               w3s, b3s)

    NP = ((N + 127) // 128) * 128
    xs = _sc_gather(x2, src2d)
    axf, axr = _run_tc(edges2, xs, weights)
    partials = _sc_scatter(axf, axr, dstf, dstr,
                           jnp.zeros((NP, D), jnp.float32))
    return _run_add(partials, N)


# speedup vs baseline: 6.1731x; 1.0048x over previous
import functools

import jax
import jax.numpy as jnp
from jax import lax
from jax.experimental import pallas as pl
from jax.experimental.pallas import tpu as pltpu
from jax.experimental.pallas import tpu_sc as plsc

D_MODEL = 128
KSZ = 8
D_EDGE = 16
PAD_VAL = -999.0
_INV_SQRT_K = 1.0 / (KSZ ** 0.5)

_BE = 2000
_CH = 128
_PREC = lax.Precision.DEFAULT


def _shift_lanes(v, s):
    n = v.shape[1]
    if s == 0:
        return v
    if s < 0:
        z = jnp.zeros((v.shape[0], -s), v.dtype)
        return jnp.concatenate([z, v[:, : n + s]], axis=1)
    z = jnp.zeros((v.shape[0], s), v.dtype)
    return jnp.concatenate([v[:, s:], z], axis=1)


def _tc_body(edges_ref, xf_ref, xr_ref,
             w0_ref, b0_ref, g0_ref, be0_ref,
             w1_ref, b1_ref, g1_ref, be1_ref,
             w2_ref, b2_ref, g2_ref, be2_ref,
             w3s_ref, b3s_ref,
             axf_ref, axr_ref):
    h = edges_ref[...]
    for w_ref, b_ref, g_ref, be_ref in (
            (w0_ref, b0_ref, g0_ref, be0_ref),
            (w1_ref, b1_ref, g1_ref, be1_ref),
            (w2_ref, b2_ref, g2_ref, be2_ref)):
        h = jnp.dot(h.astype(jnp.bfloat16), w_ref[...].astype(jnp.bfloat16),
                    precision=_PREC,
                    preferred_element_type=jnp.float32) + b_ref[...]
        h = jnp.maximum(h, 0.0)
        mu = jnp.mean(h, axis=-1, keepdims=True)
        var = jnp.mean((h - mu) ** 2, axis=-1, keepdims=True)
        v = var + 1e-5
        r = lax.rsqrt(v)
        r = r * (1.5 - 0.5 * v * r * r)
        h = (h - mu) * r * g_ref[...] + be_ref[...]

    hb = h.astype(jnp.bfloat16)
    xf = xf_ref[...]
    xr = xr_ref[...]
    axf = jnp.zeros_like(xf)
    axr = jnp.zeros_like(xr)
    for k in range(KSZ):
        a_k = (jnp.dot(hb, w3s_ref[k].astype(jnp.bfloat16), precision=_PREC,
                       preferred_element_type=jnp.float32)
               + b3s_ref[k:k + 1, :]) * _INV_SQRT_K
        s = k - KSZ // 2
        axf = axf + _shift_lanes(xf, s) * a_k
        axr = axr + _shift_lanes(xr, s) * a_k
    mask = edges_ref[:, 0:1] == PAD_VAL
    axf_ref[...] = jnp.where(mask, 0.0, axf)
    axr_ref[...] = jnp.where(mask, 0.0, axr)


def _tc_specs(E, BE):
    nb = E // BE
    full = lambda *dims: pl.BlockSpec(dims, lambda i: (0,) * len(dims))
    in_specs = [
        pl.BlockSpec((BE, D_EDGE), lambda i: (i, 0)),
        pl.BlockSpec((BE, D_MODEL), lambda i: (i, 0)),
        pl.BlockSpec((BE, D_MODEL), lambda i: (i + nb, 0)),
        full(D_EDGE, D_MODEL), full(1, D_MODEL), full(1, D_MODEL), full(1, D_MODEL),
        full(D_MODEL, D_MODEL), full(1, D_MODEL), full(1, D_MODEL), full(1, D_MODEL),
        full(D_MODEL, D_MODEL), full(1, D_MODEL), full(1, D_MODEL), full(1, D_MODEL),
        full(KSZ, D_MODEL, D_MODEL), full(KSZ, D_MODEL),
    ]
    out_specs = [
        pl.BlockSpec((BE, D_MODEL), lambda i: (i, 0)),
        pl.BlockSpec((BE, D_MODEL), lambda i: (i, 0)),
    ]
    out_shape = [
        jax.ShapeDtypeStruct((E, D_MODEL), jnp.float32),
        jax.ShapeDtypeStruct((E, D_MODEL), jnp.float32),
    ]
    return dict(grid=(nb,), in_specs=in_specs, out_specs=out_specs,
                out_shape=out_shape)


def _run_tc(edges2, xs, weights):
    E = edges2.shape[0]
    sp = _tc_specs(E, _BE)
    return pl.pallas_call(
        _tc_body,
        grid=sp["grid"],
        in_specs=sp["in_specs"],
        out_specs=sp["out_specs"],
        out_shape=sp["out_shape"],
    )(edges2, xs, xs, *weights)


def _sc_mesh():
    return plsc.VectorSubcoreMesh(core_axis_name="core",
                                  subcore_axis_name="subcore")


def _cast_body(x_ref, o_ref):
    o_ref[...] = x_ref[...].astype(jnp.bfloat16)


def _run_cast(x2, BN=2000):
    N, D = x2.shape
    return pl.pallas_call(
        _cast_body,
        grid=(N // BN,),
        in_specs=[pl.BlockSpec((BN, D), lambda i: (i, 0))],
        out_specs=pl.BlockSpec((BN, D), lambda i: (i, 0)),
        out_shape=jax.ShapeDtypeStruct((N, D), jnp.bfloat16),
    )(x2)


def _sc_gather(x2, src2d):
    M = src2d.shape[1]

    @functools.partial(
        pl.kernel,
        out_type=jax.ShapeDtypeStruct((M, D_MODEL), jnp.float32),
        mesh=_sc_mesh(),
    )
    def k(x_hbm, i_hbm, o_hbm):
        def body(i_vmem, o_vmem):
            pltpu.sync_copy(x_hbm.at[i_vmem.at[0]], o_vmem)

        pltpu.emit_pipeline(
            body,
            grid=(M // _CH,),
            in_specs=[pl.BlockSpec((1, _CH), lambda i: (0, i))],
            out_specs=[pl.BlockSpec((_CH, D_MODEL), lambda i: (i, 0))],
            core_axis_name=("core", "subcore"),
            dimension_semantics=(pltpu.PARALLEL,),
        )(i_hbm, o_hbm)

    return k(x2, src2d)


def _sc_scatter(axf, axr, dstf, dstr, zeros_nd):
    E = axf.shape[0]
    NP = zeros_nd.shape[0]
    n_sub = 16
    rp = NP // n_sub

    @functools.partial(
        pl.kernel,
        out_type=jax.ShapeDtypeStruct((2, NP, D_MODEL), jnp.float32),
        mesh=_sc_mesh(),
        scratch_types=[pltpu.VMEM_SHARED((NP, D_MODEL), jnp.float32)],
    )
    def k(axf_hbm, axr_hbm, dstf_hbm, dstr_hbm, z_hbm, o_hbm, m_sh):
        c = lax.axis_index("core")
        s = lax.axis_index("subcore")
        pltpu.sync_copy(z_hbm.at[pl.ds(s * rp, rp)],
                        m_sh.at[pl.ds(s * rp, rp)])
        plsc.subcore_barrier()

        def body(ax_vmem, i_vmem):
            pltpu.sync_copy(ax_vmem, m_sh.at[i_vmem.at[0]], add=True)

        for ax_hbm, dst_hbm in ((axf_hbm, dstf_hbm), (axr_hbm, dstr_hbm)):
            pltpu.emit_pipeline(
                body,
                grid=(E // _CH,),
                in_specs=[pl.BlockSpec((_CH, D_MODEL), lambda i: (i, 0)),
                          pl.BlockSpec((1, _CH), lambda i: (0, i))],
                out_specs=[],
                core_axis_name=("core", "subcore"),
                dimension_semantics=(pltpu.PARALLEL,),
            )(ax_hbm, dst_hbm)
        plsc.subcore_barrier()
        pltpu.sync_copy(m_sh.at[pl.ds(s * rp, rp)],
                        o_hbm.at[c, pl.ds(s * rp, rp)])

    return k(axf, axr, dstf, dstr, zeros_nd)


def _add_body(p_ref, o_ref):
    o_ref[0] = p_ref[0] + p_ref[1]


def _run_add(partials, N, BN=2000):
    _, NP, D = partials.shape
    return pl.pallas_call(
        _add_body,
        grid=(N // BN,),
        in_specs=[pl.BlockSpec((2, BN, D), lambda i: (0, i, 0))],
        out_specs=pl.BlockSpec((1, BN, D), lambda i: (0, i, 0)),
        out_shape=jax.ShapeDtypeStruct((1, N, D), jnp.float32),
    )(partials)


def kernel(x, edges, pairs_idx,
           W0, b0, g0, be0, W1, b1, g1, be1, W2, b2, g2, be2, W3, b3):
    B, N, D = x.shape
    _, E, _ = edges.shape
    x2 = x[0]
    edges2 = edges[0]
    p = pairs_idx[0]

    src2d = jnp.concatenate([p[:, 1], p[:, 0]]).reshape(1, 2 * E)
    dstf = p[:, 0].reshape(1, E)
    dstr = p[:, 1].reshape(1, E)

    w3s = jnp.transpose(W3.reshape(D, D, KSZ), (2, 0, 1))
    b3s = jnp.transpose(b3.reshape(D, KSZ), (1, 0))
    weights = (W0, b0.reshape(1, D), g0.reshape(1, D), be0.reshape(1, D),
               W1, b1.reshape(1, D), g1.reshape(1, D), be1.reshape(1, D),
               W2, b2.reshape(1, D), g2.reshape(1, D), be2.reshape(1, D),
               w3s, b3s)

    NP = ((N + 127) // 128) * 128
    xs = _sc_gather(x2, src2d)
    axf, axr = _run_tc(edges2, xs, weights)
    partials = _sc_scatter(axf, axr, dstf, dstr,
                           jnp.zeros((NP, D), jnp.float32))
    return _run_add(partials, N)


# speedup vs baseline: 6.6882x; 1.0834x over previous
import functools

import jax
import jax.numpy as jnp
from jax import lax
from jax.experimental import pallas as pl
from jax.experimental.pallas import tpu as pltpu
from jax.experimental.pallas import tpu_sc as plsc

D_MODEL = 128
KSZ = 8
D_EDGE = 16
PAD_VAL = -999.0
_INV_SQRT_K = 1.0 / (KSZ ** 0.5)

_BE = 2000
_CH = 128


def _dot(a, b):
    return jnp.dot(a, b, preferred_element_type=jnp.float32)


def _kh_body(edges_ref,
             w0_ref, b0_ref, g0_ref, be0_ref,
             w1_ref, b1_ref, g1_ref, be1_ref,
             w2_ref, b2_ref, g2_ref, be2_ref,
             h_ref):
    h = edges_ref[...]
    for w_ref, b_ref, g_ref, be_ref in (
            (w0_ref, b0_ref, g0_ref, be0_ref),
            (w1_ref, b1_ref, g1_ref, be1_ref),
            (w2_ref, b2_ref, g2_ref, be2_ref)):
        h = _dot(h.astype(jnp.bfloat16), w_ref[...]) + b_ref[...]
        h = jnp.maximum(h, 0.0)
        mu = jnp.mean(h, axis=-1, keepdims=True)
        var = jnp.mean((h - mu) ** 2, axis=-1, keepdims=True)
        v = var + 1e-5
        r = lax.rsqrt(v)
        r = r * (1.5 - 0.5 * v * r * r)
        h = (h - mu) * r * g_ref[...] + be_ref[...]
    h_ref[...] = h.astype(jnp.bfloat16)


def _run_kh(edges2, weights):
    E = edges2.shape[0]
    nb = E // _BE
    full = lambda *dims: pl.BlockSpec(dims, lambda i: (0,) * len(dims))
    return pl.pallas_call(
        _kh_body,
        grid=(nb,),
        in_specs=[
            pl.BlockSpec((_BE, D_EDGE), lambda i: (i, 0)),
            full(D_EDGE, D_MODEL), full(1, D_MODEL), full(1, D_MODEL), full(1, D_MODEL),
            full(D_MODEL, D_MODEL), full(1, D_MODEL), full(1, D_MODEL), full(1, D_MODEL),
            full(D_MODEL, D_MODEL), full(1, D_MODEL), full(1, D_MODEL), full(1, D_MODEL),
        ],
        out_specs=pl.BlockSpec((_BE, D_MODEL), lambda i: (i, 0)),
        out_shape=jax.ShapeDtypeStruct((E, D_MODEL), jnp.bfloat16),
    )(edges2, *weights)


def _shift_lanes(v, s):
    n = v.shape[1]
    if s == 0:
        return v
    if s < 0:
        z = jnp.zeros((v.shape[0], -s), v.dtype)
        return jnp.concatenate([z, v[:, : n + s]], axis=1)
    z = jnp.zeros((v.shape[0], s), v.dtype)
    return jnp.concatenate([v[:, s:], z], axis=1)


def _kax_body(edges_ref, h_ref, xf_ref, xr_ref, w3p_ref, b3p_ref,
              axf_ref, axr_ref):
    a = _dot(h_ref[...], w3p_ref[...]) + b3p_ref[...]
    xf = xf_ref[...]
    xr = xr_ref[...]
    axf = jnp.zeros_like(xf)
    axr = jnp.zeros_like(xr)
    for k in range(KSZ):
        a_k = a[:, k * D_MODEL:(k + 1) * D_MODEL]
        s = k - KSZ // 2
        axf = axf + _shift_lanes(xf, s) * a_k
        axr = axr + _shift_lanes(xr, s) * a_k
    mask = edges_ref[:, 0:1] == PAD_VAL
    axf_ref[...] = jnp.where(mask, 0.0, axf)
    axr_ref[...] = jnp.where(mask, 0.0, axr)


def _run_kax(edges2, h_all, xs, w3p, b3p):
    E = edges2.shape[0]
    nb = E // _BE
    full = lambda *dims: pl.BlockSpec(dims, lambda i: (0,) * len(dims))
    return pl.pallas_call(
        _kax_body,
        grid=(nb,),
        in_specs=[
            pl.BlockSpec((_BE, D_EDGE), lambda i: (i, 0)),
            pl.BlockSpec((_BE, D_MODEL), lambda i: (i, 0)),
            pl.BlockSpec((_BE, D_MODEL), lambda i: (i, 0)),
            pl.BlockSpec((_BE, D_MODEL), lambda i: (i + nb, 0)),
            full(D_MODEL, KSZ * D_MODEL), full(1, KSZ * D_MODEL),
        ],
        out_specs=[
            pl.BlockSpec((_BE, D_MODEL), lambda i: (i, 0)),
            pl.BlockSpec((_BE, D_MODEL), lambda i: (i, 0)),
        ],
        out_shape=[
            jax.ShapeDtypeStruct((E, D_MODEL), jnp.float32),
            jax.ShapeDtypeStruct((E, D_MODEL), jnp.float32),
        ],
    )(edges2, h_all, xs, xs, w3p, b3p)


def _sc_mesh():
    return plsc.VectorSubcoreMesh(core_axis_name="core",
                                  subcore_axis_name="subcore")


def _sc_gather(x2, src2d):
    M = src2d.shape[1]

    @functools.partial(
        pl.kernel,
        out_type=jax.ShapeDtypeStruct((M, D_MODEL), jnp.float32),
        mesh=_sc_mesh(),
    )
    def k(x_hbm, i_hbm, o_hbm):
        def body(i_vmem, o_vmem):
            pltpu.sync_copy(x_hbm.at[i_vmem.at[0]], o_vmem)

        pltpu.emit_pipeline(
            body,
            grid=(M // _CH,),
            in_specs=[pl.BlockSpec((1, _CH), lambda i: (0, i))],
            out_specs=[pl.BlockSpec((_CH, D_MODEL), lambda i: (i, 0))],
            core_axis_name=("core", "subcore"),
            dimension_semantics=(pltpu.PARALLEL,),
        )(i_hbm, o_hbm)

    return k(x2, src2d)


def _sc_scatter(axf, axr, dstf, dstr, zeros_nd):
    E = axf.shape[0]
    NP = zeros_nd.shape[0]
    n_sub = 16
    rp = NP // n_sub

    @functools.partial(
        pl.kernel,
        out_type=jax.ShapeDtypeStruct((2, NP, D_MODEL), jnp.float32),
        mesh=_sc_mesh(),
        scratch_types=[pltpu.VMEM_SHARED((NP, D_MODEL), jnp.float32)],
    )
    def k(axf_hbm, axr_hbm, dstf_hbm, dstr_hbm, z_hbm, o_hbm, m_sh):
        c = lax.axis_index("core")
        s = lax.axis_index("subcore")
        pltpu.sync_copy(z_hbm.at[pl.ds(s * rp, rp)],
                        m_sh.at[pl.ds(s * rp, rp)])
        plsc.subcore_barrier()

        def body(ax_vmem, i_vmem):
            pltpu.sync_copy(ax_vmem, m_sh.at[i_vmem.at[0]], add=True)

        for ax_hbm, dst_hbm in ((axf_hbm, dstf_hbm), (axr_hbm, dstr_hbm)):
            pltpu.emit_pipeline(
                body,
                grid=(E // _CH,),
                in_specs=[pl.BlockSpec((_CH, D_MODEL), lambda i: (i, 0)),
                          pl.BlockSpec((1, _CH), lambda i: (0, i))],
                out_specs=[],
                core_axis_name=("core", "subcore"),
                dimension_semantics=(pltpu.PARALLEL,),
            )(ax_hbm, dst_hbm)
        plsc.subcore_barrier()
        pltpu.sync_copy(m_sh.at[pl.ds(s * rp, rp)],
                        o_hbm.at[c, pl.ds(s * rp, rp)])

    return k(axf, axr, dstf, dstr, zeros_nd)


def _add_body(p_ref, o_ref):
    o_ref[0] = p_ref[0] + p_ref[1]


def _run_add(partials, N, BN=2000):
    _, NP, D = partials.shape
    return pl.pallas_call(
        _add_body,
        grid=(N // BN,),
        in_specs=[pl.BlockSpec((2, BN, D), lambda i: (0, i, 0))],
        out_specs=pl.BlockSpec((1, BN, D), lambda i: (0, i, 0)),
        out_shape=jax.ShapeDtypeStruct((1, N, D), jnp.float32),
    )(partials)


def kernel(x, edges, pairs_idx,
           W0, b0, g0, be0, W1, b1, g1, be1, W2, b2, g2, be2, W3, b3):
    B, N, D = x.shape
    _, E, _ = edges.shape
    x2 = x[0]
    edges2 = edges[0]
    p = pairs_idx[0]

    src2d = jnp.concatenate([p[:, 1], p[:, 0]]).reshape(1, 2 * E)
    dstf = p[:, 0].reshape(1, E)
    dstr = p[:, 1].reshape(1, E)

    w3p = jnp.transpose(W3.reshape(D, D, KSZ), (0, 2, 1)).reshape(D, KSZ * D)
    w3p = (w3p * _INV_SQRT_K).astype(jnp.bfloat16)
    b3p = (jnp.transpose(b3.reshape(D, KSZ), (1, 0)).reshape(1, KSZ * D)
           * _INV_SQRT_K)
    weights = (W0.astype(jnp.bfloat16), b0.reshape(1, D), g0.reshape(1, D), be0.reshape(1, D),
               W1.astype(jnp.bfloat16), b1.reshape(1, D), g1.reshape(1, D), be1.reshape(1, D),
               W2.astype(jnp.bfloat16), b2.reshape(1, D), g2.reshape(1, D), be2.reshape(1, D))

    NP = ((N + 127) // 128) * 128
    h_all = _run_kh(edges2, weights)
    xs = _sc_gather(x2, src2d)
    axf, axr = _run_kax(edges2, h_all, xs, w3p, b3p)
    partials = _sc_scatter(axf, axr, dstf, dstr,
                           jnp.zeros((NP, D), jnp.float32))
    return _run_add(partials, N)
